# baseline (speedup 1.0000x reference)
import dataclasses
import functools

import jax
import jax.numpy as jnp
from jax import lax
from jax.experimental import pallas as pl
from jax.experimental.pallas import tpu as pltpu
from jax.experimental.pallas import tpu_sc as plsc

N = 10000
E = 320000
D = 128
NP = 10240

NC = 2
NS = 16
NW = NC * NS
CHUNK = 128
EPW = 10240
NCH = EPW // CHUNK
EPAD = EPW * NW
RPS = NP // NS

_mesh = plsc.VectorSubcoreMesh(core_axis_name="c", subcore_axis_name="s")

_cp_no_layout = pltpu.CompilerParams()
if "needs_layout_passes" in pltpu.CompilerParams.__dataclass_fields__:
    _cp_no_layout = dataclasses.replace(_cp_no_layout, needs_layout_passes=False)


@functools.partial(
    pl.kernel,
    out_type=(jax.ShapeDtypeStruct((NC, NP, D), jnp.float32),
              jax.ShapeDtypeStruct((NC, NP // CHUNK, CHUNK), jnp.float32)),
    mesh=_mesh,
    scratch_types=[
        pltpu.VMEM((CHUNK,), jnp.int32),
        pltpu.VMEM((CHUNK,), jnp.int32),
        pltpu.VMEM((CHUNK,), jnp.int32),
        pltpu.VMEM((CHUNK,), jnp.int32),
        pltpu.VMEM((CHUNK,), jnp.int32),
        pltpu.VMEM((CHUNK,), jnp.int32),
        pltpu.VMEM((CHUNK, D), jnp.float32),
        pltpu.VMEM((CHUNK, D), jnp.float32),
        pltpu.VMEM((NP // CHUNK, CHUNK), jnp.float32),
        pltpu.VMEM((NP // CHUNK,), jnp.int32),
        pltpu.VMEM_SHARED((NP, D), jnp.float32),
        pltpu.VMEM_SHARED((NP // CHUNK, CHUNK), jnp.float32),
        pltpu.SemaphoreType.DMA,
        pltpu.SemaphoreType.DMA,
        pltpu.SemaphoreType.DMA,
        pltpu.SemaphoreType.DMA,
        pltpu.SemaphoreType.DMA,
        pltpu.SemaphoreType.DMA,
        pltpu.SemaphoreType.DMA,
        pltpu.SemaphoreType.DMA,
    ],
    compiler_params=_cp_no_layout,
)
def _agg(x_hbm, src_hbm, dst_hbm, z_hbm, i80_hbm, out_hbm, hist_hbm,
         sv0, sv1, sv2, sv3, dv0, dv1, r0, r1, hist_v, i80_v, acc_sh,
         cnt_sh, ss0, ss1, ss2, ss3, ds0, ds1, gs0, gs1):
    c = lax.axis_index("c")
    s = lax.axis_index("s")
    wid = s * NC + c
    sv = (sv0, sv1, sv2, sv3)
    dv = (dv0, dv1)
    rows = (r0, r1)
    ss = (ss0, ss1, ss2, ss3)
    ds = (ds0, ds1)
    gs = (gs0, gs1)
    ones16 = jnp.ones((16,), jnp.float32)

    def fetch_src(q, m):
        pltpu.async_copy(src_hbm.at[wid, q], sv[m], ss[m])

    def fetch_dst(q, b):
        pltpu.async_copy(dst_hbm.at[wid, pl.ds(q * CHUNK, CHUNK)],
                         dv[b], ds[b])

    def issue_gather(q, m, b):
        pltpu.make_async_copy(src_hbm.at[wid, 0], sv[m], ss[m]).wait()
        pltpu.async_copy(x_hbm.at[sv[m]], rows[b], gs[b])

    def consume(q, b):
        pltpu.make_async_copy(x_hbm.at[sv[0]], rows[b], gs[b]).wait()
        pltpu.make_async_copy(dst_hbm.at[wid, pl.ds(0, CHUNK)],
                              dv[b], ds[b]).wait()
        pltpu.sync_copy(rows[b], acc_sh.at[dv[b]], add=True)
        for t in range(CHUNK // 16):
            idx = dv[b][pl.ds(t * 16, 16)]
            plsc.addupdate_scatter(
                hist_v, [idx >> 7, idx & 127], ones16)

    for m in range(4):
        fetch_src(m, m)
    fetch_dst(0, 0)
    fetch_dst(1, 1)

    @pl.loop(0, NP // CHUNK)
    def _(i):
        for t in range(CHUNK // 16):
            hist_v[i, pl.ds(t * 16, 16)] = jnp.zeros((16,), jnp.float32)

    pltpu.sync_copy(i80_hbm, i80_v)
    pltpu.sync_copy(z_hbm.at[pl.ds(s * RPS, RPS)],
                    acc_sh.at[pl.ds(s * RPS, RPS)])

    @pl.when(s == 0)
    def _():
        pltpu.sync_copy(z_hbm.at[pl.ds(0, NP // CHUNK)], cnt_sh)

    plsc.subcore_barrier()
    issue_gather(0, 0, 0)
    issue_gather(1, 1, 1)

    @pl.loop(0, NCH - 4, step=4)
    def _(j):
        for k in range(4):
            q = j + k
            b = k % 2
            consume(q, b)
            fetch_src(q + 4, k)
            fetch_dst(q + 2, b)
            issue_gather(q + 2, (k + 2) % 4, b)

    for k in range(4):
        q = NCH - 4 + k
        b = k % 2
        consume(q, b)
        if k < 2:
            fetch_dst(q + 2, b)
            issue_gather(q + 2, (k + 2) % 4, b)

    pltpu.sync_copy(hist_v, cnt_sh.at[i80_v], add=True)
    plsc.subcore_barrier()

    @pl.when(s == 0)
    def _():
        pltpu.sync_copy(cnt_sh, hist_hbm.at[c])

    pltpu.sync_copy(acc_sh.at[pl.ds(s * RPS, RPS)],
                    out_hbm.at[c, pl.ds(s * RPS, RPS)])


BT = 2000


def _dot_t(a, w):
    return lax.dot_general(a, w, (((1,), (1,)), ((), ())),
                           precision=lax.Precision.HIGHEST,
                           preferred_element_type=jnp.float32)


def _tc_self(x_ref, wr_ref, b_ref, o_ref):
    o_ref[...] = _dot_t(x_ref[...], wr_ref[...]) + b_ref[...]


def _tc_layer(acc_ref, cnt_ref, self_ref, wl_ref, o_ref, *, relu):
    ssum = acc_ref[0] + acc_ref[1]
    cnt = cnt_ref[0] + cnt_ref[1]
    agg = ssum * (1.0 / jnp.maximum(cnt, 1.0))
    o = _dot_t(agg, wl_ref[...]) + self_ref[...]
    o_ref[...] = jnp.maximum(o, 0.0) if relu else o


_spec_rows = pl.BlockSpec((BT, D), lambda i: (i, 0))
_spec_acc = pl.BlockSpec((NC, BT, D), lambda i: (0, i, 0))
_spec_cnt = pl.BlockSpec((NC, BT, 1), lambda i: (0, i, 0))
_spec_w = pl.BlockSpec((D, D), lambda i: (0, 0))
_spec_b = pl.BlockSpec((1, D), lambda i: (0, 0))


def _tc_self_call(rows, wr, b):
    return pl.pallas_call(
        _tc_self,
        grid=(N // BT,),
        in_specs=[_spec_rows, _spec_w, _spec_b],
        out_specs=_spec_rows,
        out_shape=jax.ShapeDtypeStruct((N, D), jnp.float32),
    )(rows, wr, b.reshape(1, D))


def _tc_layer_call(acc, cnt, selfp, wl, relu):
    return pl.pallas_call(
        functools.partial(_tc_layer, relu=relu),
        grid=(N // BT,),
        in_specs=[_spec_acc, _spec_cnt, _spec_rows, _spec_w],
        out_specs=_spec_rows,
        out_shape=jax.ShapeDtypeStruct((N, D), jnp.float32),
    )(acc, cnt, selfp, wl)


@jax.jit
def kernel(x, edge_index, W1l, b1l, W1r, W2l, b2l, W2r):
    src = edge_index[0].astype(jnp.int32)
    dst = edge_index[1].astype(jnp.int32)
    npad = EPAD - E
    pad_src = jnp.arange(npad, dtype=jnp.int32) % N
    pad_dst = N + jnp.arange(npad, dtype=jnp.int32) % (NP - N)
    src_r = jnp.concatenate([src, pad_src]).reshape(NW, NCH, CHUNK)
    dst_r = jnp.concatenate([dst, pad_dst]).reshape(NW, EPW)

    z_d = jnp.zeros((NP, D), jnp.float32)

    i80 = jnp.arange(NP // CHUNK, dtype=jnp.int32)
    acc1, hist = _agg(x, src_r, dst_r, z_d, i80)
    cnt = hist.reshape(NC, NP, 1)
    self1 = _tc_self_call(x, W1r, b1l)
    h = _tc_layer_call(acc1, cnt, self1, W1l, True)

    acc2, _h2 = _agg(h, src_r, dst_r, z_d, i80)
    self2 = _tc_self_call(h, W2r, b2l)
    return _tc_layer_call(acc2, cnt, self2, W2l, False)


# speedup vs baseline: 1.0210x; 1.0210x over previous
import dataclasses
import functools

import jax
import jax.numpy as jnp
from jax import lax
from jax.experimental import pallas as pl
from jax.experimental.pallas import tpu as pltpu
from jax.experimental.pallas import tpu_sc as plsc

N = 10000
E = 320000
D = 128
NP = 10240

NC = 2
NS = 16
NW = NC * NS
CHUNK = 128
EPW = 10240
NCH = EPW // CHUNK
EPAD = EPW * NW
RPS = NP // NS

_mesh = plsc.VectorSubcoreMesh(core_axis_name="c", subcore_axis_name="s")

_cp_no_layout = pltpu.CompilerParams()
if "needs_layout_passes" in pltpu.CompilerParams.__dataclass_fields__:
    _cp_no_layout = dataclasses.replace(_cp_no_layout, needs_layout_passes=False)


@functools.partial(
    pl.kernel,
    out_type=jax.ShapeDtypeStruct((NC, NP, D), jnp.float32),
    mesh=_mesh,
    scratch_types=[
        pltpu.VMEM((NCH, CHUNK), jnp.int32),
        pltpu.VMEM((CHUNK,), jnp.int32),
        pltpu.VMEM((CHUNK,), jnp.int32),
        pltpu.VMEM((CHUNK, D), jnp.float32),
        pltpu.VMEM((CHUNK, D), jnp.float32),
        pltpu.VMEM_SHARED((NP, D), jnp.float32),
        pltpu.SemaphoreType.DMA,
        pltpu.SemaphoreType.DMA,
        pltpu.SemaphoreType.DMA,
        pltpu.SemaphoreType.DMA,
    ],
)
def _agg(x_hbm, src_hbm, dst_hbm, z_hbm, out_hbm, src_sl, dstv0, dstv1,
         rows0, rows1, acc_sh, sem0, sem1, semd0, semd1):
    c = lax.axis_index("c")
    s = lax.axis_index("s")
    wid = s * NC + c

    pltpu.sync_copy(z_hbm.at[pl.ds(s * RPS, RPS)],
                    acc_sh.at[pl.ds(s * RPS, RPS)])
    pltpu.sync_copy(src_hbm.at[wid], src_sl)
    plsc.subcore_barrier()

    pltpu.async_copy(x_hbm.at[src_sl.at[0]], rows0, sem0)
    pltpu.async_copy(dst_hbm.at[wid, pl.ds(0, CHUNK)], dstv0, semd0)
    pltpu.async_copy(x_hbm.at[src_sl.at[1]], rows1, sem1)
    pltpu.async_copy(dst_hbm.at[wid, pl.ds(CHUNK, CHUNK)], dstv1, semd1)

    @pl.loop(0, NCH - 2, step=2)
    def _(j):
        pltpu.make_async_copy(x_hbm.at[src_sl.at[j]], rows0, sem0).wait()
        pltpu.make_async_copy(dst_hbm.at[wid, pl.ds(j * CHUNK, CHUNK)], dstv0, semd0).wait()
        pltpu.sync_copy(rows0, acc_sh.at[dstv0], add=True)
        pltpu.async_copy(x_hbm.at[src_sl.at[j + 2]], rows0, sem0)
        pltpu.async_copy(dst_hbm.at[wid, pl.ds((j + 2) * CHUNK, CHUNK)], dstv0, semd0)
        pltpu.make_async_copy(x_hbm.at[src_sl.at[j + 1]], rows1, sem1).wait()
        pltpu.make_async_copy(dst_hbm.at[wid, pl.ds((j + 1) * CHUNK, CHUNK)], dstv1, semd1).wait()
        pltpu.sync_copy(rows1, acc_sh.at[dstv1], add=True)
        pltpu.async_copy(x_hbm.at[src_sl.at[j + 3]], rows1, sem1)
        pltpu.async_copy(dst_hbm.at[wid, pl.ds((j + 3) * CHUNK, CHUNK)], dstv1, semd1)

    pltpu.make_async_copy(x_hbm.at[src_sl.at[NCH - 2]], rows0, sem0).wait()
    pltpu.make_async_copy(dst_hbm.at[wid, pl.ds((NCH - 2) * CHUNK, CHUNK)], dstv0, semd0).wait()
    pltpu.sync_copy(rows0, acc_sh.at[dstv0], add=True)
    pltpu.make_async_copy(x_hbm.at[src_sl.at[NCH - 1]], rows1, sem1).wait()
    pltpu.make_async_copy(dst_hbm.at[wid, pl.ds((NCH - 1) * CHUNK, CHUNK)], dstv1, semd1).wait()
    pltpu.sync_copy(rows1, acc_sh.at[dstv1], add=True)

    plsc.subcore_barrier()
    pltpu.sync_copy(acc_sh.at[pl.ds(s * RPS, RPS)],
                    out_hbm.at[c, pl.ds(s * RPS, RPS)])


@functools.partial(
    pl.kernel,
    out_type=jax.ShapeDtypeStruct((NC, NP), jnp.float32),
    mesh=_mesh,
    scratch_types=[
        pltpu.VMEM((EPW,), jnp.int32),
        pltpu.VMEM((NP,), jnp.float32),
        pltpu.VMEM((NS, RPS), jnp.float32),
        pltpu.VMEM_SHARED((NS, NP), jnp.float32),
    ],
    compiler_params=_cp_no_layout,
)
def _count(dst_hbm, out_hbm, dst_fl, hist_v, tmp_v, stage_sh):
    c = lax.axis_index("c")
    s = lax.axis_index("s")
    wid = s * NC + c

    pltpu.sync_copy(dst_hbm.at[wid], dst_fl)

    @pl.loop(0, NP // 16)
    def _(i):
        hist_v[pl.ds(i * 16, 16)] = jnp.zeros((16,), jnp.float32)

    ones16 = jnp.ones((16,), jnp.float32)

    @pl.loop(0, EPW // 16)
    def _(i):
        plsc.addupdate_scatter(hist_v, [dst_fl[pl.ds(i * 16, 16)]], ones16)

    pltpu.sync_copy(hist_v, stage_sh.at[s])
    plsc.subcore_barrier()
    pltpu.sync_copy(stage_sh.at[pl.ds(0, NS), pl.ds(s * RPS, RPS)], tmp_v)

    @pl.loop(0, RPS // 16)
    def _(i):
        sl = pl.ds(i * 16, 16)
        v = tmp_v[0, sl]
        for t in range(1, NS):
            v = v + tmp_v[t, sl]
        hist_v[sl] = v

    pltpu.sync_copy(hist_v.at[pl.ds(0, RPS)],
                    out_hbm.at[c, pl.ds(s * RPS, RPS)])


BT = 2000


def _dot_t(a, w):
    return lax.dot_general(a, w, (((1,), (1,)), ((), ())),
                           precision=lax.Precision.HIGHEST,
                           preferred_element_type=jnp.float32)


def _tc_self(x_ref, wr_ref, b_ref, o_ref):
    o_ref[...] = _dot_t(x_ref[...], wr_ref[...]) + b_ref[...]


def _tc_layer(acc_ref, cnt_ref, self_ref, wl_ref, o_ref, *, relu):
    ssum = acc_ref[0] + acc_ref[1]
    cnt = cnt_ref[0] + cnt_ref[1]
    agg = ssum * (1.0 / jnp.maximum(cnt, 1.0))
    o = _dot_t(agg, wl_ref[...]) + self_ref[...]
    o_ref[...] = jnp.maximum(o, 0.0) if relu else o


_spec_rows = pl.BlockSpec((BT, D), lambda i: (i, 0))
_spec_acc = pl.BlockSpec((NC, BT, D), lambda i: (0, i, 0))
_spec_cnt = pl.BlockSpec((NC, BT, 1), lambda i: (0, i, 0))
_spec_w = pl.BlockSpec((D, D), lambda i: (0, 0))
_spec_b = pl.BlockSpec((1, D), lambda i: (0, 0))


def _tc_self_call(rows, wr, b):
    return pl.pallas_call(
        _tc_self,
        grid=(N // BT,),
        in_specs=[_spec_rows, _spec_w, _spec_b],
        out_specs=_spec_rows,
        out_shape=jax.ShapeDtypeStruct((N, D), jnp.float32),
    )(rows, wr, b.reshape(1, D))


def _tc_layer_call(acc, cnt, selfp, wl, relu):
    return pl.pallas_call(
        functools.partial(_tc_layer, relu=relu),
        grid=(N // BT,),
        in_specs=[_spec_acc, _spec_cnt, _spec_rows, _spec_w],
        out_specs=_spec_rows,
        out_shape=jax.ShapeDtypeStruct((N, D), jnp.float32),
    )(acc, cnt, selfp, wl)


@jax.jit
def kernel(x, edge_index, W1l, b1l, W1r, W2l, b2l, W2r):
    src = edge_index[0].astype(jnp.int32)
    dst = edge_index[1].astype(jnp.int32)
    npad = EPAD - E
    pad_src = jnp.arange(npad, dtype=jnp.int32) % N
    pad_dst = N + jnp.arange(npad, dtype=jnp.int32) % (NP - N)
    src_r = jnp.concatenate([src, pad_src]).reshape(NW, NCH, CHUNK)
    dst_r = jnp.concatenate([dst, pad_dst]).reshape(NW, EPW)

    z_d = jnp.zeros((NP, D), jnp.float32)

    cnt = _count(dst_r).reshape(NC, NP, 1)
    acc1 = _agg(x, src_r, dst_r, z_d)
    self1 = _tc_self_call(x, W1r, b1l)
    h = _tc_layer_call(acc1, cnt, self1, W1l, True)

    acc2 = _agg(h, src_r, dst_r, z_d)
    self2 = _tc_self_call(h, W2r, b2l)
    return _tc_layer_call(acc2, cnt, self2, W2l, False)
